# Initial kernel scaffold; baseline (speedup 1.0000x reference)
import jax
import jax.numpy as jnp
from jax.experimental import pallas as pl


def kernel(X, seeds):
    raise NotImplementedError("write your pallas kernel here")


# speedup vs baseline: 1.2253x; 1.2253x over previous
import functools

import jax
import jax.numpy as jnp
from jax import lax
from jax.experimental import pallas as pl
from jax.experimental.pallas import tpu as pltpu
from jax.experimental.pallas import tpu_sc as plsc

N = 65536
D = 64
K = 1024
MAX_ITER = 5
ALPHA = 0.1
GAMMA = 1.0
DENOM = 1.0 + GAMMA * ALPHA ** 2
VAR2 = GAMMA * ALPHA ** 2

BN = 1024
GE = N // BN

NC = 2
NS = 16
NW = NC * NS
ROWS_PER_TILE = N // NW
CHUNK = 128
NCHUNK = ROWS_PER_TILE // CHUNK
CW = 16


def _estep_body(x_ref, c_ref, lbl_ref, part_ref):
    x = x_ref[...]
    c = c_ref[...]
    pn = jnp.sum(x * x, axis=1, keepdims=True)
    cn = jnp.sum(c * c, axis=1)[None, :]
    dot = lax.dot_general(x, c, (((1,), (1,)), ((), ())),
                          preferred_element_type=jnp.float32)
    sims = pn + cn - 2.0 * dot
    sims = sims + (jnp.float32(VAR2) * cn) / jnp.float32(D)
    smin = jnp.min(sims, axis=1, keepdims=True)
    col = lax.broadcasted_iota(jnp.int32, sims.shape, 1)
    lbl = jnp.min(jnp.where(sims == smin, col, K), axis=1)
    lbl_ref[...] = lbl
    part_ref[0, 0, 0] = jnp.sum(smin)


def _estep(X, centers):
    return pl.pallas_call(
        _estep_body,
        grid=(GE,),
        in_specs=[
            pl.BlockSpec((BN, D), lambda i: (i, 0)),
            pl.BlockSpec((K, D), lambda i: (0, 0)),
        ],
        out_specs=[
            pl.BlockSpec((BN,), lambda i: (i,)),
            pl.BlockSpec((1, 1, 1), lambda i: (i, 0, 0),
                         memory_space=pltpu.SMEM),
        ],
        out_shape=[
            jax.ShapeDtypeStruct((N,), jnp.int32),
            jax.ShapeDtypeStruct((GE, 1, 1), jnp.float32),
        ],
        compiler_params=pltpu.CompilerParams(
            dimension_semantics=("parallel",)),
    )(X, centers)


AW = D + CW


def _mstep_sc(Xf, labels, zacc):
    mesh = plsc.VectorSubcoreMesh(core_axis_name="c", subcore_axis_name="s")

    @functools.partial(
        pl.kernel,
        mesh=mesh,
        out_type=jax.ShapeDtypeStruct((NW, K * AW), jnp.float32),
        scratch_types=[
            pltpu.VMEM((CHUNK * D,), jnp.float32),
            pltpu.VMEM((ROWS_PER_TILE,), jnp.int32),
            pltpu.VMEM((K * AW,), jnp.float32),
            pltpu.SemaphoreType.DMA,
            pltpu.SemaphoreType.DMA,
        ],
    )
    def m(xf_hbm, lbl_hbm, zacc_hbm, parts_hbm, rowbuf, lblbuf, acc,
          sem_r, sem_l):
        cid = lax.axis_index("c")
        sid = lax.axis_index("s")
        wid = cid * NS + sid
        base = wid * ROWS_PER_TILE

        cz = pltpu.async_copy(zacc_hbm, acc, sem_r)
        cl = pltpu.async_copy(lbl_hbm.at[pl.ds(base, ROWS_PER_TILE)],
                              lblbuf, sem_l)
        cz.wait()
        cl.wait()
        one16 = jnp.ones((16,), jnp.float32)

        @pl.loop(0, NCHUNK)
        def _(i):
            pltpu.async_copy(
                xf_hbm.at[pl.ds((base + i * CHUNK) * D, CHUNK * D)],
                rowbuf, sem_r,
            ).wait()

            @pl.loop(0, CHUNK, step=16)
            def _(g):
                lblvec = lblbuf[pl.ds(i * CHUNK + g, 16)]
                for r in range(16):
                    fb = lblvec[r] * AW
                    for j in range(D // 16):
                        v = rowbuf[pl.ds((g + r) * D + j * 16, 16)]
                        plsc.addupdate(acc.at[pl.ds(fb + j * 16, 16)], v)
                    plsc.addupdate(acc.at[pl.ds(fb + D, 16)], one16)

        pltpu.sync_copy(acc, parts_hbm.at[wid])

    return m(Xf, labels, zacc)


def _finalize_body(parts_ref, prev_ref, out_ref):
    red = jnp.sum(parts_ref[...], axis=0)
    sums = red[:, :D]
    counts = red[:, D:D + 1]
    means = sums / jnp.maximum(counts, 1.0)
    means = jnp.where(counts > 0, means, prev_ref[...] * jnp.float32(DENOM))
    out_ref[...] = means / jnp.float32(DENOM)


def _finalize(parts, prev_centers):
    return pl.pallas_call(
        _finalize_body,
        out_shape=jax.ShapeDtypeStruct((K, D), jnp.float32),
    )(parts, prev_centers)


def kernel(X, seeds):
    centers = X[seeds]
    Xf = X.reshape(-1)
    zacc = jnp.zeros((K * AW,), jnp.float32)
    inertia_parts = None
    labels = None
    for _ in range(MAX_ITER):
        labels, inertia_parts = _estep(X, centers)
        parts = _mstep_sc(Xf, labels, zacc).reshape(NW, K, AW)
        centers = _finalize(parts, centers)
    labels, inertia_parts = _estep(X, centers)
    inertia = jnp.sum(inertia_parts)
    return centers, labels, inertia
